# Initial kernel scaffold; baseline (speedup 1.0000x reference)
import jax
import jax.numpy as jnp
from jax.experimental import pallas as pl


def kernel(tokens, emb_table):
    raise NotImplementedError("write your pallas kernel here")


# speedup vs baseline: 1.2192x; 1.2192x over previous
import functools

import jax
import jax.numpy as jnp
from jax import lax
from jax.experimental import pallas as pl
from jax.experimental.pallas import tpu as pltpu
from jax.experimental.pallas import tpu_sc as plsc

_PAD = 1
_B_ROWS = 4
_SEQ = 2048
_NUM_IDX = _B_ROWS * _SEQ
_DIM = 1024
_NC = 2
_NS = 16
_NW = _NC * _NS
_PER_W = _NUM_IDX // _NW
_CHUNK = 64


def _positions_body(tok_ref, pos_ref):
    tok = tok_ref[...]
    mask = (tok != _PAD).astype(jnp.int32)
    col = lax.broadcasted_iota(jnp.int32, (_B_ROWS, _SEQ), 1)
    csum = mask
    shift = 1
    while shift < _SEQ:
        csum = csum + jnp.where(col >= shift, jnp.roll(csum, shift, axis=1), 0)
        shift *= 2
    pos_ref[...] = csum * mask + _PAD


def _compute_positions(tokens):
    return pl.pallas_call(
        _positions_body,
        out_shape=jax.ShapeDtypeStruct((_B_ROWS, _SEQ), jnp.int32),
    )(tokens)


def _gather_rows(emb_table, flat_idx):
    mesh = plsc.VectorSubcoreMesh(core_axis_name="c", subcore_axis_name="s")

    @functools.partial(
        pl.kernel,
        mesh=mesh,
        out_type=jax.ShapeDtypeStruct((_NUM_IDX, _DIM), jnp.float32),
        scratch_types=[
            pltpu.VMEM((_PER_W,), jnp.int32),
            pltpu.VMEM((_CHUNK, _DIM), jnp.float32),
            pltpu.SemaphoreType.DMA,
        ],
    )
    def k(table_hbm, idx_hbm, out_hbm, idx_v, rows_v, sem):
        wid = lax.axis_index("s") * _NC + lax.axis_index("c")
        base = wid * _PER_W
        pltpu.sync_copy(idx_hbm.at[pl.ds(base, _PER_W)], idx_v)

        @pl.loop(0, _PER_W, step=_CHUNK)
        def _(c):
            pltpu.async_copy(
                table_hbm.at[idx_v.at[pl.ds(c, _CHUNK)]], rows_v, sem
            ).wait()
            pltpu.sync_copy(rows_v, out_hbm.at[pl.ds(base + c, _CHUNK)])

    return k(emb_table, flat_idx)


def kernel(tokens, emb_table):
    tokens = tokens.astype(jnp.int32)
    positions = _compute_positions(tokens)
    out = _gather_rows(emb_table, positions.reshape(_NUM_IDX))
    return out.reshape(_B_ROWS, _SEQ, _DIM)
